# baseline (speedup 1.0000x reference)
import functools

import jax
import jax.numpy as jnp
from jax import lax
from jax.experimental import pallas as pl
from jax.experimental.pallas import tpu as pltpu
from jax.experimental.pallas import tpu_sc as plsc

N = 10000
E = 320000
D = 128
R = 200
NEG_SLOPE = 0.2291666667

NC = 2
NS = 16
NW = NC * NS

CHUNK = 128
E3 = 3 * E
NCHUNKS = 7680
NPAIR = NCHUNKS * CHUNK
CPT = NCHUNKS // NS
SEG = 20
NSEG = CPT // SEG
SPROWS = 10240
RPT = SPROWS // NS
TRASH = 10016
DEGB = 10080


def _tc_pre_body(dyn_ref, rel_ref, wn_ref, h_ref, hw_ref, rw_ref):
    x = dyn_ref[...]
    nrm = jnp.sqrt(jnp.sum(x * x, axis=1, keepdims=True))
    h = x / jnp.maximum(nrm, 1e-12)
    h_ref[...] = h
    hw_ref[...] = jnp.dot(h, wn_ref[...], preferred_element_type=jnp.float32)

    @pl.when(pl.program_id(0) == 0)
    def _():
        e = rel_ref[...]
        nr = jnp.sqrt(jnp.sum(e * e, axis=1, keepdims=True))
        r = e / jnp.maximum(nr, 1e-12)
        rw_ref[...] = jnp.dot(r, wn_ref[...], preferred_element_type=jnp.float32)


def _tc_pre(dyn, rel, w_neigh):
    blk = 1000
    grid = N // blk
    return pl.pallas_call(
        _tc_pre_body,
        grid=(grid,),
        in_specs=[
            pl.BlockSpec((blk, D), lambda i: (i, 0)),
            pl.BlockSpec((R, D), lambda i: (0, 0)),
            pl.BlockSpec((D, D), lambda i: (0, 0)),
        ],
        out_specs=[
            pl.BlockSpec((blk, D), lambda i: (i, 0)),
            pl.BlockSpec((blk, D), lambda i: (i, 0)),
            pl.BlockSpec((R, D), lambda i: (0, 0)),
        ],
        out_shape=[
            jax.ShapeDtypeStruct((N, D), jnp.float32),
            jax.ShapeDtypeStruct((N, D), jnp.float32),
            jax.ShapeDtypeStruct((R, D), jnp.float32),
        ],
    )(dyn, rel, w_neigh)


def _sc_agg_body(t_hbm, gidx_hbm, dst_hbm, out_agg,
                 sidx, didx, h0, h1, agg_sp, gsem0, gsem1, ssem0, ssem1):
    s = lax.axis_index("s")
    zero16 = jnp.zeros((16,), jnp.float32)

    def zrow(i, _):
        for cb in range(D // 16):
            h0[i, pl.ds(cb * 16, 16)] = zero16
        return 0
    lax.fori_loop(0, CHUNK, zrow, 0)

    base = s * RPT

    def zsp(i, _):
        pltpu.sync_copy(h0, agg_sp.at[pl.ds(base + i * CHUNK, CHUNK)])
        return 0
    lax.fori_loop(0, RPT // CHUNK, zsp, 0)

    plsc.subcore_barrier()

    def gather(j, buf, sem):
        return pltpu.async_copy(t_hbm.at[sidx.at[j]], buf, sem)

    def scatter(j, buf, sem):
        return pltpu.async_copy(buf, agg_sp.at[didx.at[j]], sem, add=True)

    def wait_gather(j, buf, sem):
        pltpu.make_async_copy(t_hbm.at[sidx.at[j]], buf, sem).wait()

    def wait_scatter(j, buf, sem):
        pltpu.make_async_copy(buf, agg_sp.at[didx.at[j]], sem).wait()

    def seg_body(g, _):
        pltpu.sync_copy(gidx_hbm.at[s * NSEG + g], sidx)
        pltpu.sync_copy(dst_hbm.at[s * NSEG + g], didx)

        gather(0, h0, gsem0)
        gather(1, h1, gsem1)

        def chunk_body(k, _):
            j = 2 * k
            wait_gather(j, h0, gsem0)
            scatter(j, h0, ssem0)
            wait_gather(j + 1, h1, gsem1)
            scatter(j + 1, h1, ssem1)
            wait_scatter(j, h0, ssem0)
            gather(j + 2, h0, gsem0)
            wait_scatter(j + 1, h1, ssem1)
            gather(j + 3, h1, gsem1)
            return 0

        lax.fori_loop(0, (SEG - 2) // 2, chunk_body, 0)

        wait_gather(SEG - 2, h0, gsem0)
        scatter(SEG - 2, h0, ssem0)
        wait_gather(SEG - 1, h1, gsem1)
        scatter(SEG - 1, h1, ssem1)
        wait_scatter(SEG - 2, h0, ssem0)
        wait_scatter(SEG - 1, h1, ssem1)
        return 0

    lax.fori_loop(0, NSEG, seg_body, 0)

    plsc.subcore_barrier()

    pltpu.sync_copy(agg_sp.at[pl.ds(base, RPT)], out_agg.at[pl.ds(base, RPT)])


def _sc_agg(table, gidx3, dst3):
    mesh = plsc.VectorSubcoreMesh(core_axis_name="c", subcore_axis_name="s",
                                  num_cores=1, num_subcores=NS)
    f = functools.partial(
        pl.kernel,
        out_type=jax.ShapeDtypeStruct((SPROWS, D), jnp.float32),
        mesh=mesh,
        scratch_types=[
            pltpu.VMEM((SEG, CHUNK), jnp.int32),
            pltpu.VMEM((SEG, CHUNK), jnp.int32),
            pltpu.VMEM((CHUNK, D), jnp.float32),
            pltpu.VMEM((CHUNK, D), jnp.float32),
            pltpu.VMEM_SHARED((SPROWS, D), jnp.float32),
            pltpu.SemaphoreType.DMA,
            pltpu.SemaphoreType.DMA,
            pltpu.SemaphoreType.DMA,
            pltpu.SemaphoreType.DMA,
        ],
    )(_sc_agg_body)
    return f(table, gidx3, dst3)


def _tc_post_body(h_ref, agg_ref, degp_ref, wl_ref,
                  wih_ref, whh_ref, bih_ref, bhh_ref, tgw_ref, tgb_ref,
                  out_ref):
    h = h_ref[...]
    deg = degp_ref[...][:, 0]
    agg = agg_ref[...] / jnp.maximum(deg, 1.0)[:, None]
    pre = agg + jnp.dot(h, wl_ref[...], preferred_element_type=jnp.float32)
    x = jnp.where(pre >= 0, pre, NEG_SLOPE * pre)
    gi = jnp.dot(x, wih_ref[...], preferred_element_type=jnp.float32) + bih_ref[...]
    gh = jnp.dot(h, whh_ref[...], preferred_element_type=jnp.float32) + bhh_ref[...]
    i_r, i_z, i_n = gi[:, :D], gi[:, D:2 * D], gi[:, 2 * D:]
    h_r, h_z, h_n = gh[:, :D], gh[:, D:2 * D], gh[:, 2 * D:]
    rg = jax.nn.sigmoid(i_r + h_r)
    zg = jax.nn.sigmoid(i_z + h_z)
    ng = jnp.tanh(i_n + rg * h_n)
    h_new = (1.0 - zg) * ng + zg * h
    gate = jax.nn.sigmoid(
        jnp.dot(h, tgw_ref[...], preferred_element_type=jnp.float32) + tgb_ref[...])
    out_ref[...] = gate * h_new + (1.0 - gate) * h


def _tc_post(h, agg, degp, w_loop, wih_t, whh_t, b_ih, b_hh, tg_w, tg_b):
    blk = 1000
    grid = N // blk
    return pl.pallas_call(
        _tc_post_body,
        grid=(grid,),
        in_specs=[
            pl.BlockSpec((blk, D), lambda i: (i, 0)),
            pl.BlockSpec((blk, D), lambda i: (i, 0)),
            pl.BlockSpec((blk, 1), lambda i: (i, 0)),
            pl.BlockSpec((D, D), lambda i: (0, 0)),
            pl.BlockSpec((D, 3 * D), lambda i: (0, 0)),
            pl.BlockSpec((D, 3 * D), lambda i: (0, 0)),
            pl.BlockSpec((1, 3 * D), lambda i: (0, 0)),
            pl.BlockSpec((1, 3 * D), lambda i: (0, 0)),
            pl.BlockSpec((D, D), lambda i: (0, 0)),
            pl.BlockSpec((1, D), lambda i: (0, 0)),
        ],
        out_specs=pl.BlockSpec((blk, D), lambda i: (i, 0)),
        out_shape=jax.ShapeDtypeStruct((N, D), jnp.float32),
    )(h, agg, degp, w_loop, wih_t, whh_t, b_ih, b_hh, tg_w, tg_b)


def kernel(edge_index, edge_type, dynamic_emb, emb_rel, W_neigh, W_loop,
           gru_w_ih, gru_w_hh, gru_b_ih, gru_b_hh, time_gate_w, time_gate_b):
    src = edge_index[0]
    dst = edge_index[1]

    h, hw, rw = _tc_pre(dynamic_emb, emb_rel, W_neigh)

    table = jnp.concatenate([hw, rw, jnp.eye(D, dtype=jnp.float32)], axis=0)
    npad = NPAIR - E3
    gmix = jnp.stack([src, edge_type + N, (dst % D) + (N + R)], axis=1).reshape(-1)
    dmix = jnp.stack([dst, dst, DEGB + dst // D], axis=1).reshape(-1)
    gidx3 = jnp.concatenate([
        gmix, jnp.zeros((npad,), jnp.int32)]).reshape(NS * NSEG, SEG, CHUNK)
    dst3 = jnp.concatenate([
        dmix, TRASH + jnp.arange(npad, dtype=jnp.int32) % 16
    ]).reshape(NS * NSEG, SEG, CHUNK)

    out_sp = _sc_agg(table, gidx3, dst3)
    agg = out_sp[:N]
    degp = out_sp[DEGB:DEGB + 80].reshape(-1)[:N].reshape(N, 1)

    out = _tc_post(h, agg, degp, W_loop,
                   gru_w_ih.T, gru_w_hh.T,
                   gru_b_ih.reshape(1, 3 * D), gru_b_hh.reshape(1, 3 * D),
                   time_gate_w, time_gate_b.reshape(1, D))
    return out


# speedup vs baseline: 1.2511x; 1.2511x over previous
import functools

import jax
import jax.numpy as jnp
from jax import lax
from jax.experimental import pallas as pl
from jax.experimental.pallas import tpu as pltpu
from jax.experimental.pallas import tpu_sc as plsc

N = 10000
E = 320000
D = 128
R = 200
NEG_SLOPE = 0.2291666667

NC = 2
NS = 16
NW = NC * NS

CHUNK = 128
E3 = 3 * E
NCHUNKS = 7680
NPAIR = NCHUNKS * CHUNK
CPT = NCHUNKS // NS
SEG = 20
NSEG = CPT // SEG
SPROWS = 10240
RPT = SPROWS // NS
TRASH = 10016
DEGB = 10080


def _tc_pre_body(dyn_ref, rel_ref, wn_ref, h_ref, hw_ref, rw_ref):
    x = dyn_ref[...]
    nrm = jnp.sqrt(jnp.sum(x * x, axis=1, keepdims=True))
    h = x / jnp.maximum(nrm, 1e-12)
    h_ref[...] = h
    hw_ref[...] = jnp.dot(h, wn_ref[...], preferred_element_type=jnp.float32)

    @pl.when(pl.program_id(0) == 0)
    def _():
        e = rel_ref[...]
        nr = jnp.sqrt(jnp.sum(e * e, axis=1, keepdims=True))
        r = e / jnp.maximum(nr, 1e-12)
        rw_ref[...] = jnp.dot(r, wn_ref[...], preferred_element_type=jnp.float32)


def _tc_pre(dyn, rel, w_neigh):
    blk = 1000
    grid = N // blk
    return pl.pallas_call(
        _tc_pre_body,
        grid=(grid,),
        in_specs=[
            pl.BlockSpec((blk, D), lambda i: (i, 0)),
            pl.BlockSpec((R, D), lambda i: (0, 0)),
            pl.BlockSpec((D, D), lambda i: (0, 0)),
        ],
        out_specs=[
            pl.BlockSpec((blk, D), lambda i: (i, 0)),
            pl.BlockSpec((blk, D), lambda i: (i, 0)),
            pl.BlockSpec((R, D), lambda i: (0, 0)),
        ],
        out_shape=[
            jax.ShapeDtypeStruct((N, D), jnp.float32),
            jax.ShapeDtypeStruct((N, D), jnp.float32),
            jax.ShapeDtypeStruct((R, D), jnp.float32),
        ],
    )(dyn, rel, w_neigh)


def _sc_agg_body(t_hbm, gidx_hbm, dst_hbm, out_agg,
                 sidx, didx, h0, h1, agg_sp, gsem0, gsem1, ssem0, ssem1):
    s = lax.axis_index("s")
    zero16 = jnp.zeros((16,), jnp.float32)

    def zrow(i, _):
        for cb in range(D // 16):
            h0[i, pl.ds(cb * 16, 16)] = zero16
        return 0
    lax.fori_loop(0, CHUNK, zrow, 0)

    base = s * RPT

    def zsp(i, _):
        pltpu.sync_copy(h0, agg_sp.at[pl.ds(base + i * CHUNK, CHUNK)])
        return 0
    lax.fori_loop(0, RPT // CHUNK, zsp, 0)

    plsc.subcore_barrier()

    def gather(j, buf, sem):
        return pltpu.async_copy(t_hbm.at[sidx.at[j]], buf, sem)

    def scatter(j, buf, sem):
        return pltpu.async_copy(buf, agg_sp.at[didx.at[j]], sem, add=True)

    def wait_gather(j, buf, sem):
        pltpu.make_async_copy(t_hbm.at[sidx.at[j]], buf, sem).wait()

    def wait_scatter(j, buf, sem):
        pltpu.make_async_copy(buf, agg_sp.at[didx.at[j]], sem).wait()

    def seg_body(g, _):
        pltpu.sync_copy(gidx_hbm.at[s * NSEG + g], sidx)
        pltpu.sync_copy(dst_hbm.at[s * NSEG + g], didx)

        gather(0, h0, gsem0)
        gather(1, h1, gsem1)

        def chunk_body(k, _):
            j = 2 * k
            wait_gather(j, h0, gsem0)
            scatter(j, h0, ssem0)
            wait_gather(j + 1, h1, gsem1)
            scatter(j + 1, h1, ssem1)
            wait_scatter(j, h0, ssem0)
            gather(j + 2, h0, gsem0)
            wait_scatter(j + 1, h1, ssem1)
            gather(j + 3, h1, gsem1)
            return 0

        lax.fori_loop(0, (SEG - 2) // 2, chunk_body, 0)

        wait_gather(SEG - 2, h0, gsem0)
        scatter(SEG - 2, h0, ssem0)
        wait_gather(SEG - 1, h1, gsem1)
        scatter(SEG - 1, h1, ssem1)
        wait_scatter(SEG - 2, h0, ssem0)
        wait_scatter(SEG - 1, h1, ssem1)
        return 0

    lax.fori_loop(0, NSEG, seg_body, 0)

    plsc.subcore_barrier()

    pltpu.sync_copy(agg_sp.at[pl.ds(base, RPT)], out_agg.at[pl.ds(base, RPT)])


def _sc_agg(table, gidx3, dst3):
    mesh = plsc.VectorSubcoreMesh(core_axis_name="c", subcore_axis_name="s",
                                  num_cores=1, num_subcores=NS)
    f = functools.partial(
        pl.kernel,
        out_type=jax.ShapeDtypeStruct((SPROWS, D), jnp.float32),
        mesh=mesh,
        scratch_types=[
            pltpu.VMEM((SEG, CHUNK), jnp.int32),
            pltpu.VMEM((SEG, CHUNK), jnp.int32),
            pltpu.VMEM((CHUNK, D), jnp.float32),
            pltpu.VMEM((CHUNK, D), jnp.float32),
            pltpu.VMEM_SHARED((SPROWS, D), jnp.float32),
            pltpu.SemaphoreType.DMA,
            pltpu.SemaphoreType.DMA,
            pltpu.SemaphoreType.DMA,
            pltpu.SemaphoreType.DMA,
        ],
    )(_sc_agg_body)
    return f(table, gidx3, dst3)


def _tc_post_body(h_ref, agg_ref, degp_ref, wl_ref,
                  wih_ref, whh_ref, bih_ref, bhh_ref, tgw_ref, tgb_ref,
                  out_ref):
    h = h_ref[...]
    deg = degp_ref[...][:, 0]
    agg = agg_ref[...] / jnp.maximum(deg, 1.0)[:, None]
    pre = agg + jnp.dot(h, wl_ref[...], preferred_element_type=jnp.float32)
    x = jnp.where(pre >= 0, pre, NEG_SLOPE * pre)
    gi = jnp.dot(x, wih_ref[...], preferred_element_type=jnp.float32) + bih_ref[...]
    gh = jnp.dot(h, whh_ref[...], preferred_element_type=jnp.float32) + bhh_ref[...]
    i_r, i_z, i_n = gi[:, :D], gi[:, D:2 * D], gi[:, 2 * D:]
    h_r, h_z, h_n = gh[:, :D], gh[:, D:2 * D], gh[:, 2 * D:]
    rg = jax.nn.sigmoid(i_r + h_r)
    zg = jax.nn.sigmoid(i_z + h_z)
    ng = jnp.tanh(i_n + rg * h_n)
    h_new = (1.0 - zg) * ng + zg * h
    gate = jax.nn.sigmoid(
        jnp.dot(h, tgw_ref[...], preferred_element_type=jnp.float32) + tgb_ref[...])
    out_ref[...] = gate * h_new + (1.0 - gate) * h


def _tc_post(h, agg, degp, w_loop, wih_t, whh_t, b_ih, b_hh, tg_w, tg_b):
    blk = 1000
    grid = N // blk
    return pl.pallas_call(
        _tc_post_body,
        grid=(grid,),
        in_specs=[
            pl.BlockSpec((blk, D), lambda i: (i, 0)),
            pl.BlockSpec((blk, D), lambda i: (i, 0)),
            pl.BlockSpec((blk, 1), lambda i: (i, 0)),
            pl.BlockSpec((D, D), lambda i: (0, 0)),
            pl.BlockSpec((D, 3 * D), lambda i: (0, 0)),
            pl.BlockSpec((D, 3 * D), lambda i: (0, 0)),
            pl.BlockSpec((1, 3 * D), lambda i: (0, 0)),
            pl.BlockSpec((1, 3 * D), lambda i: (0, 0)),
            pl.BlockSpec((D, D), lambda i: (0, 0)),
            pl.BlockSpec((1, D), lambda i: (0, 0)),
        ],
        out_specs=pl.BlockSpec((blk, D), lambda i: (i, 0)),
        out_shape=jax.ShapeDtypeStruct((N, D), jnp.float32),
    )(h, agg, degp, w_loop, wih_t, whh_t, b_ih, b_hh, tg_w, tg_b)


def kernel(edge_index, edge_type, dynamic_emb, emb_rel, W_neigh, W_loop,
           gru_w_ih, gru_w_hh, gru_b_ih, gru_b_hh, time_gate_w, time_gate_b):
    src = edge_index[0]
    dst = edge_index[1]

    h, hw, rw = _tc_pre(dynamic_emb, emb_rel, W_neigh)

    table = jnp.concatenate([hw, rw, jnp.eye(D, dtype=jnp.float32)], axis=0)
    npad = NPAIR - E3
    ept = E // NS
    ppt = npad // NS
    slab_g = jnp.concatenate([
        src.reshape(NS, ept),
        (edge_type + N).reshape(NS, ept),
        ((dst % D) + (N + R)).reshape(NS, ept),
        jnp.zeros((NS, ppt), jnp.int32)], axis=1)
    trash = TRASH + (jnp.arange(ppt, dtype=jnp.int32) % 16)
    slab_d = jnp.concatenate([
        dst.reshape(NS, ept),
        dst.reshape(NS, ept),
        (DEGB + dst // D).reshape(NS, ept),
        jnp.broadcast_to(trash, (NS, ppt))], axis=1)
    gidx3 = slab_g.reshape(NS * NSEG, SEG, CHUNK)
    dst3 = slab_d.reshape(NS * NSEG, SEG, CHUNK)

    out_sp = _sc_agg(table, gidx3, dst3)
    agg = out_sp[:N]
    degp = out_sp[DEGB:DEGB + 80].reshape(-1)[:N].reshape(N, 1)

    out = _tc_post(h, agg, degp, W_loop,
                   gru_w_ih.T, gru_w_hh.T,
                   gru_b_ih.reshape(1, 3 * D), gru_b_hh.reshape(1, 3 * D),
                   time_gate_w, time_gate_b.reshape(1, D))
    return out


# speedup vs baseline: 1.2610x; 1.0079x over previous
import functools

import jax
import jax.numpy as jnp
from jax import lax
from jax.experimental import pallas as pl
from jax.experimental.pallas import tpu as pltpu
from jax.experimental.pallas import tpu_sc as plsc

N = 10000
E = 320000
D = 128
R = 200
NEG_SLOPE = 0.2291666667

NC = 2
NS = 16
NW = NC * NS

CHUNK = 128
E3 = 3 * E
NCHUNKS = 7680
NPAIR = NCHUNKS * CHUNK
CPT = NCHUNKS // NS
SEG = 40
NSEG = CPT // SEG
SPROWS = 10240
RPT = SPROWS // NS
TRASH = 10000
DEGB = 10016


def _tc_pre_body(dyn_ref, rel_ref, wn_ref, h_ref, hw_ref, rw_ref):
    x = dyn_ref[...]
    nrm = jnp.sqrt(jnp.sum(x * x, axis=1, keepdims=True))
    h = x / jnp.maximum(nrm, 1e-12)
    h_ref[...] = h
    hw_ref[...] = jnp.dot(h, wn_ref[...], preferred_element_type=jnp.float32)

    @pl.when(pl.program_id(0) == 0)
    def _():
        e = rel_ref[...]
        nr = jnp.sqrt(jnp.sum(e * e, axis=1, keepdims=True))
        r = e / jnp.maximum(nr, 1e-12)
        rw_ref[...] = jnp.dot(r, wn_ref[...], preferred_element_type=jnp.float32)


def _tc_pre(dyn, rel, w_neigh):
    blk = 1000
    grid = N // blk
    return pl.pallas_call(
        _tc_pre_body,
        grid=(grid,),
        in_specs=[
            pl.BlockSpec((blk, D), lambda i: (i, 0)),
            pl.BlockSpec((R, D), lambda i: (0, 0)),
            pl.BlockSpec((D, D), lambda i: (0, 0)),
        ],
        out_specs=[
            pl.BlockSpec((blk, D), lambda i: (i, 0)),
            pl.BlockSpec((blk, D), lambda i: (i, 0)),
            pl.BlockSpec((R, D), lambda i: (0, 0)),
        ],
        out_shape=[
            jax.ShapeDtypeStruct((N, D), jnp.float32),
            jax.ShapeDtypeStruct((N, D), jnp.float32),
            jax.ShapeDtypeStruct((R, D), jnp.float32),
        ],
    )(dyn, rel, w_neigh)


def _sc_agg_body(t_hbm, gidx_hbm, dst_hbm, out_agg,
                 sidx, didx, h0, h1, agg_sp, gsem0, gsem1, ssem0, ssem1):
    s = lax.axis_index("s")
    zero16 = jnp.zeros((16,), jnp.float32)

    def zrow(i, _):
        for cb in range(D // 16):
            h0[i, pl.ds(cb * 16, 16)] = zero16
        return 0
    lax.fori_loop(0, CHUNK, zrow, 0)

    base = s * RPT

    def zsp(i, _):
        pltpu.sync_copy(h0, agg_sp.at[pl.ds(base + i * CHUNK, CHUNK)])
        return 0
    lax.fori_loop(0, RPT // CHUNK, zsp, 0)

    plsc.subcore_barrier()

    def gather(j, buf, sem):
        return pltpu.async_copy(t_hbm.at[sidx.at[j]], buf, sem)

    def scatter(j, buf, sem):
        return pltpu.async_copy(buf, agg_sp.at[didx.at[j]], sem, add=True)

    def wait_gather(j, buf, sem):
        pltpu.make_async_copy(t_hbm.at[sidx.at[j]], buf, sem).wait()

    def wait_scatter(j, buf, sem):
        pltpu.make_async_copy(buf, agg_sp.at[didx.at[j]], sem).wait()

    def seg_body(g, _):
        pltpu.sync_copy(gidx_hbm.at[s * NSEG + g], sidx)
        pltpu.sync_copy(dst_hbm.at[s * NSEG + g], didx)

        gather(0, h0, gsem0)
        gather(1, h1, gsem1)

        def chunk_body(k, _):
            j = 2 * k
            wait_gather(j, h0, gsem0)
            scatter(j, h0, ssem0)
            wait_gather(j + 1, h1, gsem1)
            scatter(j + 1, h1, ssem1)
            wait_scatter(j, h0, ssem0)
            gather(j + 2, h0, gsem0)
            wait_scatter(j + 1, h1, ssem1)
            gather(j + 3, h1, gsem1)
            return 0

        lax.fori_loop(0, (SEG - 2) // 2, chunk_body, 0)

        wait_gather(SEG - 2, h0, gsem0)
        scatter(SEG - 2, h0, ssem0)
        wait_gather(SEG - 1, h1, gsem1)
        scatter(SEG - 1, h1, ssem1)
        wait_scatter(SEG - 2, h0, ssem0)
        wait_scatter(SEG - 1, h1, ssem1)
        return 0

    lax.fori_loop(0, NSEG, seg_body, 0)

    plsc.subcore_barrier()

    pltpu.sync_copy(agg_sp.at[pl.ds(base, RPT)], out_agg.at[pl.ds(base, RPT)])


def _sc_agg(table, gidx3, dst3):
    mesh = plsc.VectorSubcoreMesh(core_axis_name="c", subcore_axis_name="s",
                                  num_cores=1, num_subcores=NS)
    f = functools.partial(
        pl.kernel,
        out_type=jax.ShapeDtypeStruct((SPROWS, D), jnp.float32),
        mesh=mesh,
        scratch_types=[
            pltpu.VMEM((SEG, CHUNK), jnp.int32),
            pltpu.VMEM((SEG, CHUNK), jnp.int32),
            pltpu.VMEM((CHUNK, D), jnp.float32),
            pltpu.VMEM((CHUNK, D), jnp.float32),
            pltpu.VMEM_SHARED((SPROWS, D), jnp.float32),
            pltpu.SemaphoreType.DMA,
            pltpu.SemaphoreType.DMA,
            pltpu.SemaphoreType.DMA,
            pltpu.SemaphoreType.DMA,
        ],
    )(_sc_agg_body)
    return f(table, gidx3, dst3)


def _tc_post_body(h_ref, agg_ref, degp_ref, wl_ref,
                  wih_ref, whh_ref, bih_ref, bhh_ref, tgw_ref, tgb_ref,
                  out_ref):
    h = h_ref[...]
    deg = degp_ref[...][:, 0]
    agg = agg_ref[...] / jnp.maximum(deg, 1.0)[:, None]
    pre = agg + jnp.dot(h, wl_ref[...], preferred_element_type=jnp.float32)
    x = jnp.where(pre >= 0, pre, NEG_SLOPE * pre)
    gi = jnp.dot(x, wih_ref[...], preferred_element_type=jnp.float32) + bih_ref[...]
    gh = jnp.dot(h, whh_ref[...], preferred_element_type=jnp.float32) + bhh_ref[...]
    i_r, i_z, i_n = gi[:, :D], gi[:, D:2 * D], gi[:, 2 * D:]
    h_r, h_z, h_n = gh[:, :D], gh[:, D:2 * D], gh[:, 2 * D:]
    rg = jax.nn.sigmoid(i_r + h_r)
    zg = jax.nn.sigmoid(i_z + h_z)
    ng = jnp.tanh(i_n + rg * h_n)
    h_new = (1.0 - zg) * ng + zg * h
    gate = jax.nn.sigmoid(
        jnp.dot(h, tgw_ref[...], preferred_element_type=jnp.float32) + tgb_ref[...])
    out_ref[...] = gate * h_new + (1.0 - gate) * h


def _tc_post(h, agg, degp, w_loop, wih_t, whh_t, b_ih, b_hh, tg_w, tg_b):
    blk = 1000
    grid = N // blk
    return pl.pallas_call(
        _tc_post_body,
        grid=(grid,),
        in_specs=[
            pl.BlockSpec((blk, D), lambda i: (i, 0)),
            pl.BlockSpec((blk, D), lambda i: (i, 0)),
            pl.BlockSpec((blk, 1), lambda i: (i, 0)),
            pl.BlockSpec((D, D), lambda i: (0, 0)),
            pl.BlockSpec((D, 3 * D), lambda i: (0, 0)),
            pl.BlockSpec((D, 3 * D), lambda i: (0, 0)),
            pl.BlockSpec((1, 3 * D), lambda i: (0, 0)),
            pl.BlockSpec((1, 3 * D), lambda i: (0, 0)),
            pl.BlockSpec((D, D), lambda i: (0, 0)),
            pl.BlockSpec((1, D), lambda i: (0, 0)),
        ],
        out_specs=pl.BlockSpec((blk, D), lambda i: (i, 0)),
        out_shape=jax.ShapeDtypeStruct((N, D), jnp.float32),
    )(h, agg, degp, w_loop, wih_t, whh_t, b_ih, b_hh, tg_w, tg_b)


def kernel(edge_index, edge_type, dynamic_emb, emb_rel, W_neigh, W_loop,
           gru_w_ih, gru_w_hh, gru_b_ih, gru_b_hh, time_gate_w, time_gate_b):
    src = edge_index[0]
    dst = edge_index[1]

    h, hw, rw = _tc_pre(dynamic_emb, emb_rel, W_neigh)

    table = jnp.concatenate([hw, rw, jnp.eye(D, dtype=jnp.float32)], axis=0)
    npad = NPAIR - E3
    ept = E // NS
    ppt = npad // NS
    slab_g = jnp.concatenate([
        src.reshape(NS, ept),
        (edge_type + N).reshape(NS, ept),
        ((dst % D) + (N + R)).reshape(NS, ept),
        jnp.zeros((NS, ppt), jnp.int32)], axis=1)
    trash = TRASH + (jnp.arange(ppt, dtype=jnp.int32) % 16)
    slab_d = jnp.concatenate([
        dst.reshape(NS, ept),
        dst.reshape(NS, ept),
        (DEGB + dst // D).reshape(NS, ept),
        jnp.broadcast_to(trash, (NS, ppt))], axis=1)
    gidx3 = slab_g.reshape(NS * NSEG, SEG, CHUNK)
    dst3 = slab_d.reshape(NS * NSEG, SEG, CHUNK)

    out_sp = _sc_agg(table, gidx3, dst3)
    agg = out_sp[:N]
    degp = out_sp[DEGB:DEGB + 80].reshape(-1)[:N].reshape(N, 1)

    out = _tc_post(h, agg, degp, W_loop,
                   gru_w_ih.T, gru_w_hh.T,
                   gru_b_ih.reshape(1, 3 * D), gru_b_hh.reshape(1, 3 * D),
                   time_gate_w, time_gate_b.reshape(1, D))
    return out
